# Initial kernel scaffold; baseline (speedup 1.0000x reference)
import functools
import jax
import jax.numpy as jnp
from jax import lax
from jax.experimental import pallas as pl
from jax.experimental.pallas import tpu as pltpu
from jax.experimental.pallas import tpu_sc as plsc

N = 10000
E = 320000
D = 128
LAYERS = 3

NC = 2
NS = 16
NW = NC * NS
FPT = D // NW
CHUNK = 2000
NBLK = CHUNK // 16
NCHUNK = E // CHUNK
NVEC = N // 16


def _body(xt_hbm, w_hbm, ei_hbm, out_hbm, xa, xb, s, src_b, dst_b, w_b):
    cid = lax.axis_index("c")
    sid = lax.axis_index("s")
    wid = cid * NS + sid
    f0 = wid * FPT

    pltpu.sync_copy(xt_hbm.at[pl.ds(f0, FPT), :], xa)

    def init_s(i, _):
        b = i * 16
        for f in range(FPT):
            s[f, pl.ds(b, 16)] = xa[f, pl.ds(b, 16)]
        return 0
    lax.fori_loop(0, NVEC, init_s, 0)

    zeros16 = jnp.zeros((16,), jnp.float32)
    fidx = [jnp.full((16,), f, jnp.int32) for f in range(FPT)]

    for layer in range(LAYERS):
        x, y = (xa, xb) if layer % 2 == 0 else (xb, xa)

        def zero_y(i, _):
            b = i * 16
            for f in range(FPT):
                y[f, pl.ds(b, 16)] = zeros16
            return 0
        lax.fori_loop(0, NVEC, zero_y, 0)

        def do_chunk(c, _):
            e0 = pl.multiple_of(c * CHUNK, CHUNK)
            pltpu.sync_copy(ei_hbm.at[1, pl.ds(e0, CHUNK)], src_b)
            pltpu.sync_copy(ei_hbm.at[0, pl.ds(e0, CHUNK)], dst_b)
            pltpu.sync_copy(w_hbm.at[pl.ds(e0, CHUNK)], w_b)

            def do_block(j, _):
                b = j * 16
                src = src_b[pl.ds(b, 16)]
                dst = dst_b[pl.ds(b, 16)]
                wv = w_b[pl.ds(b, 16)]
                for f in range(FPT):
                    xv = plsc.load_gather(x, [fidx[f], src])
                    plsc.addupdate_scatter(y, [fidx[f], dst], xv * wv)
                return 0
            lax.fori_loop(0, NBLK, do_block, 0)
            return 0
        lax.fori_loop(0, NCHUNK, do_chunk, 0)

        def add_s(i, _):
            b = i * 16
            for f in range(FPT):
                s[f, pl.ds(b, 16)] = s[f, pl.ds(b, 16)] + y[f, pl.ds(b, 16)]
            return 0
        lax.fori_loop(0, NVEC, add_s, 0)

    quarter = jnp.full((16,), 0.25, jnp.float32)

    def finish(i, _):
        b = i * 16
        for f in range(FPT):
            s[f, pl.ds(b, 16)] = s[f, pl.ds(b, 16)] * quarter
        return 0
    lax.fori_loop(0, NVEC, finish, 0)

    pltpu.sync_copy(s, out_hbm.at[pl.ds(f0, FPT), :])


@jax.jit
def _run(xt, edge_values, edge_index):
    mesh = plsc.VectorSubcoreMesh(
        core_axis_name="c", subcore_axis_name="s",
        num_cores=NC, num_subcores=NS)
    k = pl.kernel(
        _body,
        out_type=jax.ShapeDtypeStruct((D, N), jnp.float32),
        mesh=mesh,
        scratch_types=[
            pltpu.VMEM((FPT, N), jnp.float32),
            pltpu.VMEM((FPT, N), jnp.float32),
            pltpu.VMEM((FPT, N), jnp.float32),
            pltpu.VMEM((CHUNK,), jnp.int32),
            pltpu.VMEM((CHUNK,), jnp.int32),
            pltpu.VMEM((CHUNK,), jnp.float32),
        ],
    )
    return k(xt, edge_values, edge_index)


def kernel(item_embeddings, edge_values, edge_index):
    xt = jnp.transpose(item_embeddings)
    out_t = _run(xt, edge_values, edge_index)
    return jnp.transpose(out_t)


# speedup vs baseline: 1.7840x; 1.7840x over previous
import jax
import jax.numpy as jnp
from jax import lax
from jax.experimental import pallas as pl
from jax.experimental.pallas import tpu as pltpu
from jax.experimental.pallas import tpu_sc as plsc

N = 10000
E = 320000
D = 128
LAYERS = 3

NC = 2
NS = 16
NW = NC * NS
FPT = D // NW
FN = FPT * N
CHUNK = 2000
NBLK = CHUNK // 16
NCHUNK = E // CHUNK
NVEC = FN // 16


def _body(xt_hbm, w_hbm, src_hbm, dst_hbm, out_hbm, xa, xb, s, src_b, dst_b, w_b):
    cid = lax.axis_index("c")
    sid = lax.axis_index("s")
    wid = cid * NS + sid
    base = wid * FN

    pltpu.sync_copy(xt_hbm.at[pl.ds(base, FN)], xa)

    def init_s(i, _):
        b = i * 16
        s[pl.ds(b, 16)] = xa[pl.ds(b, 16)]
        return 0
    lax.fori_loop(0, NVEC, init_s, 0)

    zeros16 = jnp.zeros((16,), jnp.float32)
    foff = [jnp.full((16,), f * N, jnp.int32) for f in range(FPT)]

    for layer in range(LAYERS):
        x, y = (xa, xb) if layer % 2 == 0 else (xb, xa)

        def zero_y(i, _):
            b = i * 16
            y[pl.ds(b, 16)] = zeros16
            return 0
        lax.fori_loop(0, NVEC, zero_y, 0)

        def do_chunk(c, _):
            e0 = pl.multiple_of(c * CHUNK, CHUNK)
            pltpu.sync_copy(src_hbm.at[pl.ds(e0, CHUNK)], src_b)
            pltpu.sync_copy(dst_hbm.at[pl.ds(e0, CHUNK)], dst_b)
            pltpu.sync_copy(w_hbm.at[pl.ds(e0, CHUNK)], w_b)

            def do_block(j, _):
                b = j * 16
                src = src_b[pl.ds(b, 16)]
                dst = dst_b[pl.ds(b, 16)]
                wv = w_b[pl.ds(b, 16)]
                for f in range(FPT):
                    xv = plsc.load_gather(x, [src + foff[f]])
                    plsc.addupdate_scatter(y, [dst + foff[f]], xv * wv)
                return 0
            lax.fori_loop(0, NBLK, do_block, 0)
            return 0
        lax.fori_loop(0, NCHUNK, do_chunk, 0)

        def add_s(i, _):
            b = i * 16
            s[pl.ds(b, 16)] = s[pl.ds(b, 16)] + y[pl.ds(b, 16)]
            return 0
        lax.fori_loop(0, NVEC, add_s, 0)

    quarter = jnp.full((16,), 0.25, jnp.float32)

    def finish(i, _):
        b = i * 16
        s[pl.ds(b, 16)] = s[pl.ds(b, 16)] * quarter
        return 0
    lax.fori_loop(0, NVEC, finish, 0)

    pltpu.sync_copy(s, out_hbm.at[pl.ds(base, FN)])


@jax.jit
def _run(xt_flat, edge_values, src, dst):
    mesh = plsc.VectorSubcoreMesh(
        core_axis_name="c", subcore_axis_name="s",
        num_cores=NC, num_subcores=NS)
    k = pl.kernel(
        _body,
        out_type=jax.ShapeDtypeStruct((D * N,), jnp.float32),
        mesh=mesh,
        compiler_params=pltpu.CompilerParams(needs_layout_passes=False),
        scratch_types=[
            pltpu.VMEM((FN,), jnp.float32),
            pltpu.VMEM((FN,), jnp.float32),
            pltpu.VMEM((FN,), jnp.float32),
            pltpu.VMEM((CHUNK,), jnp.int32),
            pltpu.VMEM((CHUNK,), jnp.int32),
            pltpu.VMEM((CHUNK,), jnp.float32),
        ],
    )
    return k(xt_flat, edge_values, src, dst)


def kernel(item_embeddings, edge_values, edge_index):
    xt_flat = jnp.transpose(item_embeddings).reshape(D * N)
    src = edge_index[1]
    dst = edge_index[0]
    out_flat = _run(xt_flat, edge_values, src, dst)
    return jnp.transpose(out_flat.reshape(D, N))


# speedup vs baseline: 2.5777x; 1.4449x over previous
import jax
import jax.numpy as jnp
from jax import lax
from jax.experimental import pallas as pl
from jax.experimental.pallas import tpu as pltpu
from jax.experimental.pallas import tpu_sc as plsc

N = 10000
E = 320000
D = 128
LAYERS = 3

NC = 2
NS = 16
NW = NC * NS
FPT = D // NW
FN = FPT * N
CHUNK = 4000
NBLK = CHUNK // 16
NCHUNK = E // CHUNK
NBUF = 4
NVEC = FN // 16
UNROLL = 2


def _body(xt_hbm, ed_hbm, out_hbm, stage_hbm, xa, xb, *rest):
    edb = rest[:NBUF]
    sems = rest[NBUF:]
    cid = lax.axis_index("c")
    sid = lax.axis_index("s")
    wid = cid * NS + sid
    base = wid * FN

    pltpu.sync_copy(xt_hbm.at[pl.ds(base, FN)], xa)

    zeros16 = jnp.zeros((16,), jnp.float32)
    lo16 = jnp.full((16,), 0xFFFF, jnp.int32)

    def process_chunk(eb):
        @pl.loop(0, NBLK // UNROLL)
        def _(j):
            for u in range(UNROLL):
                b = (j * UNROLL + u) * 16
                pk = eb[pl.ds(b, 16)]
                wv = plsc.bitcast(eb[pl.ds(CHUNK + b, 16)], jnp.float32)
                src = pk & lo16
                dst = pk >> 16
                for f in range(FPT):
                    xf = x.at[pl.ds(f * N, N)]
                    yf = y.at[pl.ds(f * N, N)]
                    xv = plsc.load_gather(xf, [src])
                    plsc.addupdate_scatter(yf, [dst], xv * wv)

    for layer in range(LAYERS):
        x, y = (xa, xb) if layer % 2 == 0 else (xb, xa)

        @pl.loop(0, NVEC // 4)
        def _(i):
            b = i * 64
            for u in range(4):
                y[pl.ds(b + u * 16, 16)] = zeros16

        for b in range(NBUF):
            pltpu.async_copy(
                ed_hbm.at[pl.ds(b * 2 * CHUNK, 2 * CHUNK)], edb[b], sems[b])

        @pl.loop(0, NCHUNK - NBUF, step=NBUF)
        def _(c4):
            for b in range(NBUF):
                cc = c4 + b
                e0 = pl.multiple_of(cc * 2 * CHUNK, 2 * CHUNK)
                pltpu.make_async_copy(
                    ed_hbm.at[pl.ds(e0, 2 * CHUNK)], edb[b], sems[b]).wait()
                process_chunk(edb[b])
                e1 = pl.multiple_of((cc + NBUF) * 2 * CHUNK, 2 * CHUNK)
                pltpu.async_copy(
                    ed_hbm.at[pl.ds(e1, 2 * CHUNK)], edb[b], sems[b])

        for b in range(NBUF):
            cc = NCHUNK - NBUF + b
            pltpu.make_async_copy(
                ed_hbm.at[pl.ds(cc * 2 * CHUNK, 2 * CHUNK)],
                edb[b], sems[b]).wait()
            process_chunk(edb[b])

        if layer == 1:
            pltpu.sync_copy(x, stage_hbm.at[pl.ds(base, FN)])

    @pl.loop(0, NVEC)
    def _(i):
        b = i * 16
        xb[pl.ds(b, 16)] = xb[pl.ds(b, 16)] + xa[pl.ds(b, 16)]

    pltpu.sync_copy(stage_hbm.at[pl.ds(base, FN)], xa)

    @pl.loop(0, NVEC)
    def _(i):
        b = i * 16
        xb[pl.ds(b, 16)] = xb[pl.ds(b, 16)] + xa[pl.ds(b, 16)]

    pltpu.sync_copy(xt_hbm.at[pl.ds(base, FN)], xa)
    quarter = jnp.full((16,), 0.25, jnp.float32)

    @pl.loop(0, NVEC)
    def _(i):
        b = i * 16
        xb[pl.ds(b, 16)] = (xb[pl.ds(b, 16)] + xa[pl.ds(b, 16)]) * quarter

    pltpu.sync_copy(xb, out_hbm.at[pl.ds(base, FN)])


@jax.jit
def _run(xt_flat, edata):
    mesh = plsc.VectorSubcoreMesh(
        core_axis_name="c", subcore_axis_name="s",
        num_cores=NC, num_subcores=NS)
    k = pl.kernel(
        _body,
        out_type=(
            jax.ShapeDtypeStruct((D * N,), jnp.float32),
            jax.ShapeDtypeStruct((D * N,), jnp.float32),
        ),
        mesh=mesh,
        compiler_params=pltpu.CompilerParams(needs_layout_passes=False),
        scratch_types=[
            pltpu.VMEM((FN,), jnp.float32),
            pltpu.VMEM((FN,), jnp.float32),
        ] + [pltpu.VMEM((2 * CHUNK,), jnp.int32)] * NBUF
          + [pltpu.SemaphoreType.DMA] * NBUF,
    )
    out_flat, _ = k(xt_flat, edata)
    return out_flat


def kernel(item_embeddings, edge_values, edge_index):
    xt_flat = jnp.transpose(item_embeddings).reshape(D * N)
    src = edge_index[1]
    dst = edge_index[0]
    pk = (dst << 16) | src
    wbits = lax.bitcast_convert_type(edge_values, jnp.int32)
    edata = jnp.concatenate(
        [pk.reshape(NCHUNK, CHUNK), wbits.reshape(NCHUNK, CHUNK)], axis=1
    ).reshape(2 * E)
    out_flat = _run(xt_flat, edata)
    return jnp.transpose(out_flat.reshape(D, N))


# speedup vs baseline: 2.5953x; 1.0068x over previous
import jax
import jax.numpy as jnp
from jax import lax
from jax.experimental import pallas as pl
from jax.experimental.pallas import tpu as pltpu
from jax.experimental.pallas import tpu_sc as plsc

N = 10000
E = 320000
D = 128
LAYERS = 3

NC = 2
NS = 16
NW = NC * NS
FPT = D // NW
FN = FPT * N
CHUNK = 4000
NBLK = CHUNK // 16
NCHUNK = E // CHUNK
NBUF = 4
NVEC = FN // 16
UNROLL = 4


def _body(xt_hbm, ed_hbm, out_hbm, stage_hbm, xa, xb, *rest):
    edb = rest[:NBUF]
    sems = rest[NBUF:]
    cid = lax.axis_index("c")
    sid = lax.axis_index("s")
    wid = cid * NS + sid
    base = wid * FN

    pltpu.sync_copy(xt_hbm.at[pl.ds(base, FN)], xa)

    zeros16 = jnp.zeros((16,), jnp.float32)
    lo16 = jnp.full((16,), 0xFFFF, jnp.int32)

    def process_chunk(eb):
        @pl.loop(0, NBLK // UNROLL)
        def _(j):
            for u in range(UNROLL):
                b = (j * UNROLL + u) * 16
                pk = eb[pl.ds(b, 16)]
                wv = plsc.bitcast(eb[pl.ds(CHUNK + b, 16)], jnp.float32)
                src = pk & lo16
                dst = pk >> 16
                for f in range(FPT):
                    xf = x.at[pl.ds(f * N, N)]
                    yf = y.at[pl.ds(f * N, N)]
                    xv = plsc.load_gather(xf, [src])
                    plsc.addupdate_scatter(yf, [dst], xv * wv)

    for layer in range(LAYERS):
        x, y = (xa, xb) if layer % 2 == 0 else (xb, xa)

        @pl.loop(0, NVEC // 4)
        def _(i):
            b = i * 64
            for u in range(4):
                y[pl.ds(b + u * 16, 16)] = zeros16

        for b in range(NBUF):
            pltpu.async_copy(
                ed_hbm.at[pl.ds(b * 2 * CHUNK, 2 * CHUNK)], edb[b], sems[b])

        @pl.loop(0, NCHUNK - NBUF, step=NBUF)
        def _(c4):
            for b in range(NBUF):
                cc = c4 + b
                e0 = pl.multiple_of(cc * 2 * CHUNK, 2 * CHUNK)
                pltpu.make_async_copy(
                    ed_hbm.at[pl.ds(e0, 2 * CHUNK)], edb[b], sems[b]).wait()
                process_chunk(edb[b])
                e1 = pl.multiple_of((cc + NBUF) * 2 * CHUNK, 2 * CHUNK)
                pltpu.async_copy(
                    ed_hbm.at[pl.ds(e1, 2 * CHUNK)], edb[b], sems[b])

        for b in range(NBUF):
            cc = NCHUNK - NBUF + b
            pltpu.make_async_copy(
                ed_hbm.at[pl.ds(cc * 2 * CHUNK, 2 * CHUNK)],
                edb[b], sems[b]).wait()
            process_chunk(edb[b])

        if layer == 1:
            pltpu.sync_copy(x, stage_hbm.at[pl.ds(base, FN)])

    @pl.loop(0, NVEC)
    def _(i):
        b = i * 16
        xb[pl.ds(b, 16)] = xb[pl.ds(b, 16)] + xa[pl.ds(b, 16)]

    pltpu.sync_copy(stage_hbm.at[pl.ds(base, FN)], xa)

    @pl.loop(0, NVEC)
    def _(i):
        b = i * 16
        xb[pl.ds(b, 16)] = xb[pl.ds(b, 16)] + xa[pl.ds(b, 16)]

    pltpu.sync_copy(xt_hbm.at[pl.ds(base, FN)], xa)
    quarter = jnp.full((16,), 0.25, jnp.float32)

    @pl.loop(0, NVEC)
    def _(i):
        b = i * 16
        xb[pl.ds(b, 16)] = (xb[pl.ds(b, 16)] + xa[pl.ds(b, 16)]) * quarter

    pltpu.sync_copy(xb, out_hbm.at[pl.ds(base, FN)])


@jax.jit
def _run(xt_flat, edata):
    mesh = plsc.VectorSubcoreMesh(
        core_axis_name="c", subcore_axis_name="s",
        num_cores=NC, num_subcores=NS)
    k = pl.kernel(
        _body,
        out_type=(
            jax.ShapeDtypeStruct((D * N,), jnp.float32),
            jax.ShapeDtypeStruct((D * N,), jnp.float32),
        ),
        mesh=mesh,
        compiler_params=pltpu.CompilerParams(needs_layout_passes=False),
        scratch_types=[
            pltpu.VMEM((FN,), jnp.float32),
            pltpu.VMEM((FN,), jnp.float32),
        ] + [pltpu.VMEM((2 * CHUNK,), jnp.int32)] * NBUF
          + [pltpu.SemaphoreType.DMA] * NBUF,
    )
    out_flat, _ = k(xt_flat, edata)
    return out_flat


def kernel(item_embeddings, edge_values, edge_index):
    xt_flat = jnp.transpose(item_embeddings).reshape(D * N)
    src = edge_index[1]
    dst = edge_index[0]
    pk = (dst << 16) | src
    wbits = lax.bitcast_convert_type(edge_values, jnp.int32)
    edata = jnp.concatenate(
        [pk.reshape(NCHUNK, CHUNK), wbits.reshape(NCHUNK, CHUNK)], axis=1
    ).reshape(2 * E)
    out_flat = _run(xt_flat, edata)
    return jnp.transpose(out_flat.reshape(D, N))
